# initial kernel scaffold (unmeasured)
import jax
import jax.numpy as jnp
from jax import lax
from jax.experimental import pallas as pl
from jax.experimental.pallas import tpu as pltpu

N_DEV = 4
S = 2048
H = 8
DH = 128
D = 1024
BLK = 64
SCALE = 0.08838834764831843
NEG = -1e9
QT = 1024


def _gather_kv_and_q(kvp, xb, wqb):

    def body(kvp_ref, xb_ref, wq_ref, kvall_ref, q_ref,
             local_sem, send_sems, recv_sems):
        my = lax.axis_index("i")
        left = lax.rem(my + N_DEV - 1, N_DEV)
        right = lax.rem(my + 1, N_DEV)

        barrier_sem = pltpu.get_barrier_semaphore()
        for nbr in (left, right):
            pl.semaphore_signal(
                barrier_sem, inc=1,
                device_id=(nbr,), device_id_type=pl.DeviceIdType.MESH,
            )
        pl.semaphore_wait(barrier_sem, 2)

        cp = pltpu.make_async_copy(kvp_ref, kvall_ref.at[my], local_sem)
        cp.start()

        qf = jnp.dot(xb_ref[...], wq_ref[...],
                     preferred_element_type=jnp.float32)
        for h in range(H):
            q_ref[h] = qf[:, h * DH:(h + 1) * DH].astype(jnp.bfloat16)

        cp.wait()

        for h in range(N_DEV - 1):
            slot = lax.rem(my - h + N_DEV, N_DEV)
            rdma = pltpu.make_async_remote_copy(
                src_ref=kvall_ref.at[slot],
                dst_ref=kvall_ref.at[slot],
                send_sem=send_sems.at[h],
                recv_sem=recv_sems.at[h],
                device_id=(right,),
                device_id_type=pl.DeviceIdType.MESH,
            )
            rdma.start()
            rdma.wait()

    return pl.pallas_call(
        body,
        out_shape=[
            jax.ShapeDtypeStruct((N_DEV, H, S, 2 * DH), jnp.bfloat16),
            jax.ShapeDtypeStruct((H, S, DH), jnp.bfloat16),
        ],
        in_specs=[
            pl.BlockSpec(memory_space=pltpu.ANY),
            pl.BlockSpec(memory_space=pltpu.VMEM),
            pl.BlockSpec(memory_space=pltpu.VMEM),
        ],
        out_specs=[
            pl.BlockSpec(memory_space=pltpu.ANY),
            pl.BlockSpec(memory_space=pltpu.VMEM),
        ],
        scratch_shapes=[
            pltpu.SemaphoreType.DMA,
            pltpu.SemaphoreType.DMA((N_DEV - 1,)),
            pltpu.SemaphoreType.DMA((N_DEV - 1,)),
        ],
        compiler_params=pltpu.CompilerParams(collective_id=0),
    )(kvp, xb, wqb)


def _attention(q, kv_all, wob):

    def body(q_ref, kvall_ref, wo_ref, out_ref,
             kvbuf, acc_ref, m_ref, l_ref, ctx_ref, dma_sem):
        my = lax.axis_index("i")

        for c in range(N_DEV):
            cp = pltpu.make_async_copy(kvall_ref.at[c], kvbuf, dma_sem)
            cp.start()
            cp.wait()
            for h in range(H):
                kh = kvbuf[h, :, 0:DH]
                vh = kvbuf[h, :, DH:2 * DH]
                for qt in range(S // QT):
                    rs = qt * QT
                    qh = q_ref[h, rs:rs + QT, :]
                    s = lax.dot_general(
                        qh, kh, (((1,), (1,)), ((), ())),
                        preferred_element_type=jnp.float32,
                    ) * SCALE
                    ir = lax.broadcasted_iota(jnp.int32, (QT, S), 0)
                    ic = lax.broadcasted_iota(jnp.int32, (QT, S), 1)
                    qb = (my * S + rs + ir) // BLK
                    kb = (c * S + ic) // BLK
                    s = jnp.where(kb <= qb, s, NEG)
                    m_cur = jnp.max(s, axis=1, keepdims=True)
                    if c == 0:
                        m_new = m_cur
                        p = jnp.exp(s - m_new)
                        l_new = jnp.sum(p, axis=1, keepdims=True)
                        acc_new = lax.dot_general(
                            p.astype(jnp.bfloat16), vh,
                            (((1,), (0,)), ((), ())),
                            preferred_element_type=jnp.float32,
                        )
                    else:
                        m_prev = m_ref[rs:rs + QT, h:h + 1]
                        l_prev = l_ref[rs:rs + QT, h:h + 1]
                        m_new = jnp.maximum(m_prev, m_cur)
                        alpha = jnp.exp(m_prev - m_new)
                        p = jnp.exp(s - m_new)
                        l_new = l_prev * alpha + jnp.sum(
                            p, axis=1, keepdims=True)
                        acc_new = acc_ref[h, rs:rs + QT, :] * alpha + (
                            lax.dot_general(
                                p.astype(jnp.bfloat16), vh,
                                (((1,), (0,)), ((), ())),
                                preferred_element_type=jnp.float32,
                            ))
                    m_ref[rs:rs + QT, h:h + 1] = m_new
                    l_ref[rs:rs + QT, h:h + 1] = l_new
                    acc_ref[h, rs:rs + QT, :] = acc_new

        for h in range(H):
            ctx_ref[:, h * DH:(h + 1) * DH] = (
                acc_ref[h] / l_ref[:, h:h + 1]).astype(jnp.bfloat16)
        out_ref[0] = jnp.dot(ctx_ref[...], wo_ref[...],
                             preferred_element_type=jnp.float32)

    return pl.pallas_call(
        body,
        out_shape=jax.ShapeDtypeStruct((1, S, D), jnp.float32),
        in_specs=[
            pl.BlockSpec(memory_space=pltpu.VMEM),
            pl.BlockSpec(memory_space=pltpu.ANY),
            pl.BlockSpec(memory_space=pltpu.VMEM),
        ],
        out_specs=pl.BlockSpec(memory_space=pltpu.VMEM),
        scratch_shapes=[
            pltpu.VMEM((H, S, 2 * DH), jnp.bfloat16),
            pltpu.VMEM((H, S, DH), jnp.float32),
            pltpu.VMEM((S, H), jnp.float32),
            pltpu.VMEM((S, H), jnp.float32),
            pltpu.VMEM((S, D), jnp.bfloat16),
            pltpu.SemaphoreType.DMA,
        ],
    )(q, kv_all, wob)


def kernel(x, Wq, K_ext, V_ext, Wo):
    xb = x[0].astype(jnp.bfloat16)
    wqb = Wq.astype(jnp.bfloat16)
    wob = Wo.astype(jnp.bfloat16)
    kvp = jnp.concatenate(
        [K_ext[0].transpose(1, 0, 2).astype(jnp.bfloat16),
         V_ext[0].transpose(1, 0, 2).astype(jnp.bfloat16)],
        axis=-1,
    )
    kv_all, q = _gather_kv_and_q(kvp, xb, wqb)
    return _attention(q, kv_all, wob)


# baseline (device time: 634015 ns/iter reference)
import jax
import jax.numpy as jnp
from jax import lax
from jax.experimental import pallas as pl
from jax.experimental.pallas import tpu as pltpu

N_DEV = 4
S = 2048
H = 8
DH = 128
D = 1024
BLK = 64
SCALE = 0.08838834764831843
NEG = -1e9
QT = 512


def _gather_kv_and_q(kvp, xb, wqb):

    def body(kvp_ref, xb_ref, wq_ref, kvall_ref, q_ref,
             local_sem, send_sems, recv_sems):
        my = lax.axis_index("i")
        left = lax.rem(my + N_DEV - 1, N_DEV)
        right = lax.rem(my + 1, N_DEV)

        barrier_sem = pltpu.get_barrier_semaphore()
        for nbr in (left, right):
            pl.semaphore_signal(
                barrier_sem, inc=1,
                device_id=(nbr,), device_id_type=pl.DeviceIdType.MESH,
            )
        pl.semaphore_wait(barrier_sem, 2)

        cp = pltpu.make_async_copy(kvp_ref, kvall_ref.at[my], local_sem)
        cp.start()

        qf = jnp.dot(xb_ref[...], wq_ref[...],
                     preferred_element_type=jnp.float32) * SCALE
        for h in range(H):
            q_ref[h] = qf[:, h * DH:(h + 1) * DH].astype(jnp.bfloat16)

        cp.wait()

        for h in range(N_DEV - 1):
            slot = lax.rem(my - h + N_DEV, N_DEV)
            rdma = pltpu.make_async_remote_copy(
                src_ref=kvall_ref.at[slot],
                dst_ref=kvall_ref.at[slot],
                send_sem=send_sems.at[h],
                recv_sem=recv_sems.at[h],
                device_id=(right,),
                device_id_type=pl.DeviceIdType.MESH,
            )
            rdma.start()
            rdma.wait()

    return pl.pallas_call(
        body,
        out_shape=[
            jax.ShapeDtypeStruct((N_DEV, H, S, 2 * DH), jnp.bfloat16),
            jax.ShapeDtypeStruct((H, S, DH), jnp.bfloat16),
        ],
        in_specs=[
            pl.BlockSpec(memory_space=pltpu.MemorySpace.HBM),
            pl.BlockSpec(memory_space=pltpu.MemorySpace.VMEM),
            pl.BlockSpec(memory_space=pltpu.MemorySpace.VMEM),
        ],
        out_specs=[
            pl.BlockSpec(memory_space=pltpu.MemorySpace.HBM),
            pl.BlockSpec(memory_space=pltpu.MemorySpace.VMEM),
        ],
        scratch_shapes=[
            pltpu.SemaphoreType.DMA,
            pltpu.SemaphoreType.DMA((N_DEV - 1,)),
            pltpu.SemaphoreType.DMA((N_DEV - 1,)),
        ],
        compiler_params=pltpu.CompilerParams(collective_id=0),
    )(kvp, xb, wqb)


def _attention(q, kv_all, wob):
    n_qt = S // QT

    def body(q_ref, kvall_ref, wo_ref, out_ref, kvbuf, dma_sems):
        my = lax.axis_index("i")
        out_ref[...] = jnp.zeros((1, S, D), jnp.float32)

        def hq_body(idx, dummy):
            h = idx // n_qt
            qt = idx % n_qt
            rs = qt * QT

            @pl.when(qt == 0)
            def _():
                cps = [
                    pltpu.make_async_copy(
                        kvall_ref.at[c, h], kvbuf.at[c], dma_sems.at[c])
                    for c in range(N_DEV)
                ]
                for cp in cps:
                    cp.start()
                for cp in cps:
                    cp.wait()

            qh = q_ref[h, pl.ds(rs, QT), :]
            qb = (my * S + rs
                  + lax.broadcasted_iota(jnp.int32, (QT, 1), 0)) // BLK

            def c_body(c, carry):
                m_prev, l_prev, acc_prev = carry
                kh = kvbuf[c, :, 0:DH]
                vh = kvbuf[c, :, DH:2 * DH]
                s = lax.dot_general(
                    qh, kh, (((1,), (1,)), ((), ())),
                    preferred_element_type=jnp.float32,
                )
                kb = (c * S
                      + lax.broadcasted_iota(jnp.int32, (1, S), 1)) // BLK
                s = jnp.where(kb <= qb, s, NEG)
                m_new = jnp.maximum(m_prev,
                                    jnp.max(s, axis=1, keepdims=True))
                alpha = jnp.exp(m_prev - m_new)
                p = jnp.exp(s - m_new)
                l_new = l_prev * alpha + jnp.sum(p, axis=1, keepdims=True)
                acc_new = acc_prev * alpha + lax.dot_general(
                    p.astype(jnp.bfloat16), vh,
                    (((1,), (0,)), ((), ())),
                    preferred_element_type=jnp.float32,
                )
                return m_new, l_new, acc_new

            m0 = jnp.full((QT, 1), -1e30, jnp.float32)
            l0 = jnp.zeros((QT, 1), jnp.float32)
            a0 = jnp.zeros((QT, DH), jnp.float32)
            m, l, acc = lax.fori_loop(0, N_DEV, c_body, (m0, l0, a0))

            ctx = (acc / l).astype(jnp.bfloat16)
            wo_h = wo_ref[pl.ds(h * DH, DH), :]
            tile = out_ref[0, pl.ds(rs, QT), :]
            out_ref[0, pl.ds(rs, QT), :] = tile + lax.dot_general(
                ctx, wo_h, (((1,), (0,)), ((), ())),
                preferred_element_type=jnp.float32,
            )
            return dummy

        lax.fori_loop(0, H * n_qt, hq_body, 0)

    return pl.pallas_call(
        body,
        out_shape=jax.ShapeDtypeStruct((1, S, D), jnp.float32),
        in_specs=[
            pl.BlockSpec(memory_space=pltpu.MemorySpace.VMEM),
            pl.BlockSpec(memory_space=pltpu.MemorySpace.HBM),
            pl.BlockSpec(memory_space=pltpu.MemorySpace.VMEM),
        ],
        out_specs=pl.BlockSpec(memory_space=pltpu.MemorySpace.VMEM),
        scratch_shapes=[
            pltpu.VMEM((N_DEV, S, 2 * DH), jnp.bfloat16),
            pltpu.SemaphoreType.DMA((N_DEV,)),
        ],
    )(q, kv_all, wob)


def kernel(x, Wq, K_ext, V_ext, Wo):
    xb = x[0].astype(jnp.bfloat16)
    wqb = Wq.astype(jnp.bfloat16)
    wob = Wo.astype(jnp.bfloat16)
    kvp = jnp.concatenate(
        [K_ext[0].transpose(1, 0, 2).astype(jnp.bfloat16),
         V_ext[0].transpose(1, 0, 2).astype(jnp.bfloat16)],
        axis=-1,
    )
    kv_all, q = _gather_kv_and_q(kvp, xb, wqb)
    return _attention(q, kv_all, wob)


# device time: 426083 ns/iter; 1.4880x vs baseline; 1.4880x over previous
import jax
import jax.numpy as jnp
from jax import lax
from jax.experimental import pallas as pl
from jax.experimental.pallas import tpu as pltpu

N_DEV = 4
S = 2048
H = 8
DH = 128
D = 1024
BLK = 64
SCALE = 0.08838834764831843
NEG = -1e9
QT = 256
N_QT = S // QT
N_HQ = H * N_QT


def _fused(kvp, xb, wqb, wob):
    def body(kvp_ref, xb_ref, wq_ref, wo_ref, out_ref,
             kvall_ref, q_ref, kvbuf, acc_ref, m_ref, l_ref,
             chunk_sem, send_sems, recv_sems):
        my = lax.axis_index("i")

        barrier_sem = pltpu.get_barrier_semaphore()
        for off in range(1, N_DEV):
            pl.semaphore_signal(
                barrier_sem, inc=1,
                device_id=(lax.rem(my + off, N_DEV),),
                device_id_type=pl.DeviceIdType.MESH,
            )
        pl.semaphore_wait(barrier_sem, N_DEV - 1)

        for off in range(1, N_DEV):
            @pl.when(my + off <= N_DEV - 1)
            def _send():
                rdma = pltpu.make_async_remote_copy(
                    src_ref=kvp_ref,
                    dst_ref=kvall_ref.at[my],
                    send_sem=send_sems.at[off - 1],
                    recv_sem=recv_sems.at[my],
                    device_id=(my + off,),
                    device_id_type=pl.DeviceIdType.MESH,
                )
                rdma.start()

        qf = jnp.dot(xb_ref[...], wq_ref[...],
                     preferred_element_type=jnp.float32) * SCALE
        for h in range(H):
            q_ref[h] = qf[:, h * DH:(h + 1) * DH].astype(jnp.bfloat16)
        m_ref[...] = jnp.full((N_HQ, 1, QT), -1e30, jnp.float32)
        l_ref[...] = jnp.zeros((N_HQ, 1, QT), jnp.float32)
        acc_ref[...] = jnp.zeros((N_HQ, DH, QT), jnp.float32)
        out_ref[...] = jnp.zeros((1, S, D), jnp.float32)

        def make_hq_body(c, masked):
            def hq_body(idx, dummy):
                h = idx // N_QT
                qt = idx % N_QT
                rs = qt * QT
                qh = q_ref[h, pl.ds(rs, QT), :]
                kh = kvbuf[h, :, 0:DH]
                vh = kvbuf[h, :, DH:2 * DH]
                sT = lax.dot_general(
                    kh, qh, (((1,), (1,)), ((), ())),
                    preferred_element_type=jnp.float32,
                )
                if masked:
                    kb = (c * S
                          + lax.broadcasted_iota(jnp.int32, (S, 1), 0)
                          ) // BLK
                    qb = (my * S + rs
                          + lax.broadcasted_iota(jnp.int32, (1, QT), 1)
                          ) // BLK
                    sT = jnp.where(kb <= qb, sT, NEG)
                m_prev = m_ref[idx]
                m_new = jnp.maximum(
                    m_prev, jnp.max(sT, axis=0, keepdims=True))
                alpha = jnp.exp(m_prev - m_new)
                p = jnp.exp(sT - m_new)
                l_ref[idx] = l_ref[idx] * alpha + jnp.sum(
                    p, axis=0, keepdims=True)
                acc_ref[idx] = acc_ref[idx] * alpha + lax.dot_general(
                    vh, p.astype(jnp.bfloat16), (((0,), (0,)), ((), ())),
                    preferred_element_type=jnp.float32,
                )
                m_ref[idx] = m_new
                return dummy
            return hq_body

        for k in range(N_DEV):
            c = my - k

            @pl.when(c >= 0)
            def _chunk():
                if k == 0:
                    cp = pltpu.make_async_copy(kvp_ref, kvbuf, chunk_sem)
                else:
                    pltpu.make_async_remote_copy(
                        src_ref=kvall_ref.at[c],
                        dst_ref=kvall_ref.at[c],
                        send_sem=send_sems.at[0],
                        recv_sem=recv_sems.at[c],
                        device_id=(my,),
                        device_id_type=pl.DeviceIdType.MESH,
                    ).wait_recv()
                    cp = pltpu.make_async_copy(
                        kvall_ref.at[c], kvbuf, chunk_sem)
                cp.start()
                cp.wait()
                lax.fori_loop(0, N_HQ, make_hq_body(c, k == 0), 0)

        def ep_body(idx, dummy):
            h = idx // N_QT
            qt = idx % N_QT
            rs = qt * QT
            ctxT = (acc_ref[idx] / l_ref[idx]).astype(jnp.bfloat16)
            wo_h = wo_ref[pl.ds(h * DH, DH), :]
            tile = lax.dot_general(
                ctxT, wo_h, (((0,), (0,)), ((), ())),
                preferred_element_type=jnp.float32,
            )
            out_ref[0, pl.ds(rs, QT), :] = (
                out_ref[0, pl.ds(rs, QT), :] + tile)
            return dummy

        lax.fori_loop(0, N_HQ, ep_body, 0)

        for off in range(1, N_DEV):
            @pl.when(my + off <= N_DEV - 1)
            def _drain():
                pltpu.make_async_remote_copy(
                    src_ref=kvp_ref,
                    dst_ref=kvall_ref.at[my],
                    send_sem=send_sems.at[off - 1],
                    recv_sem=recv_sems.at[0],
                    device_id=(my,),
                    device_id_type=pl.DeviceIdType.MESH,
                ).wait_send()

    out, _ = pl.pallas_call(
        body,
        out_shape=[
            jax.ShapeDtypeStruct((1, S, D), jnp.float32),
            jax.ShapeDtypeStruct((N_DEV, H, S, 2 * DH), jnp.bfloat16),
        ],
        in_specs=[
            pl.BlockSpec(memory_space=pltpu.MemorySpace.HBM),
            pl.BlockSpec(memory_space=pltpu.MemorySpace.VMEM),
            pl.BlockSpec(memory_space=pltpu.MemorySpace.VMEM),
            pl.BlockSpec(memory_space=pltpu.MemorySpace.VMEM),
        ],
        out_specs=[
            pl.BlockSpec(memory_space=pltpu.MemorySpace.VMEM),
            pl.BlockSpec(memory_space=pltpu.MemorySpace.HBM),
        ],
        scratch_shapes=[
            pltpu.VMEM((H, S, DH), jnp.bfloat16),
            pltpu.VMEM((H, S, 2 * DH), jnp.bfloat16),
            pltpu.VMEM((N_HQ, DH, QT), jnp.float32),
            pltpu.VMEM((N_HQ, 1, QT), jnp.float32),
            pltpu.VMEM((N_HQ, 1, QT), jnp.float32),
            pltpu.SemaphoreType.DMA,
            pltpu.SemaphoreType.DMA((N_DEV - 1,)),
            pltpu.SemaphoreType.DMA((N_DEV,)),
        ],
        compiler_params=pltpu.CompilerParams(collective_id=0),
    )(kvp, xb, wqb, wob)
    return out


def kernel(x, Wq, K_ext, V_ext, Wo):
    xb = x[0].astype(jnp.bfloat16)
    wqb = Wq.astype(jnp.bfloat16)
    wob = Wo.astype(jnp.bfloat16)
    kvp = jnp.concatenate(
        [K_ext[0].transpose(1, 0, 2).astype(jnp.bfloat16),
         V_ext[0].transpose(1, 0, 2).astype(jnp.bfloat16)],
        axis=-1,
    )
    return _fused(kvp, xb, wqb, wob)


# device time: 394177 ns/iter; 1.6085x vs baseline; 1.0809x over previous
import jax
import jax.numpy as jnp
from jax import lax
from jax.experimental import pallas as pl
from jax.experimental.pallas import tpu as pltpu

N_DEV = 4
S = 2048
H = 8
DH = 128
D = 1024
BLK = 64
SCALE = 0.08838834764831843
NEG = -1e9
QT = 256
N_QT = S // QT
N_HQ = H * N_QT


def _fused(kvp, xb, wqb, wob):
    def body(kvp_ref, xb_ref, wq_ref, wo_ref, out_ref,
             kvall_ref, q_ref, kvbuf, acc_ref, m_ref, l_ref, ctx_ref,
             chunk_sem, send_sems, recv_sems):
        my = lax.axis_index("i")

        barrier_sem = pltpu.get_barrier_semaphore()
        for off in range(1, N_DEV):
            pl.semaphore_signal(
                barrier_sem, inc=1,
                device_id=(lax.rem(my + off, N_DEV),),
                device_id_type=pl.DeviceIdType.MESH,
            )
        pl.semaphore_wait(barrier_sem, N_DEV - 1)

        for off in range(1, N_DEV):
            @pl.when(my + off <= N_DEV - 1)
            def _send():
                rdma = pltpu.make_async_remote_copy(
                    src_ref=kvp_ref,
                    dst_ref=kvall_ref.at[my],
                    send_sem=send_sems.at[off - 1],
                    recv_sem=recv_sems.at[my],
                    device_id=(my + off,),
                    device_id_type=pl.DeviceIdType.MESH,
                )
                rdma.start()

        for g in range(H // 2):
            qp = jnp.dot(xb_ref[...], wq_ref[:, g * 2 * DH:(g + 1) * 2 * DH],
                         preferred_element_type=jnp.float32) * SCALE
            q_ref[2 * g] = qp[:, 0:DH].astype(jnp.bfloat16)
            q_ref[2 * g + 1] = qp[:, DH:2 * DH].astype(jnp.bfloat16)
        m_ref[...] = jnp.full((N_HQ, 1, QT), -1e30, jnp.float32)
        l_ref[...] = jnp.zeros((N_HQ, 1, QT), jnp.float32)
        acc_ref[...] = jnp.zeros((N_HQ, DH, QT), jnp.float32)

        def flash_update(idx, qh, kh, vh, sT):
            m_prev = m_ref[idx]
            m_new = jnp.maximum(
                m_prev, jnp.max(sT, axis=0, keepdims=True))
            alpha = jnp.exp(m_prev - m_new)
            p = jnp.exp(sT - m_new)
            l_ref[idx] = l_ref[idx] * alpha + jnp.sum(
                p, axis=0, keepdims=True)
            acc_ref[idx] = acc_ref[idx] * alpha + lax.dot_general(
                vh, p.astype(jnp.bfloat16), (((0,), (0,)), ((), ())),
                preferred_element_type=jnp.float32,
            )
            m_ref[idx] = m_new

        def make_hq_body(c):
            def hq_body(idx, dummy):
                h = idx // N_QT
                qt = idx % N_QT
                rs = qt * QT
                qh = q_ref[h, pl.ds(rs, QT), :]
                kh = kvbuf[h, :, 0:DH]
                vh = kvbuf[h, :, DH:2 * DH]
                sT = lax.dot_general(
                    kh, qh, (((1,), (1,)), ((), ())),
                    preferred_element_type=jnp.float32,
                )
                flash_update(idx, qh, kh, vh, sT)
                return dummy
            return hq_body

        def make_diag_body(qt):
            rs = qt * QT
            kl = rs + QT

            def diag_body(h, dummy):
                qh = q_ref[h, rs:rs + QT, :]
                kh = kvbuf[h, 0:kl, 0:DH]
                vh = kvbuf[h, 0:kl, DH:2 * DH]
                sT = lax.dot_general(
                    kh, qh, (((1,), (1,)), ((), ())),
                    preferred_element_type=jnp.float32,
                )
                kb = lax.broadcasted_iota(jnp.int32, (kl, 1), 0) // BLK
                qb = (rs
                      + lax.broadcasted_iota(jnp.int32, (1, QT), 1)
                      ) // BLK
                sT = jnp.where(kb <= qb, sT, NEG)
                flash_update(h * N_QT + qt, qh, kh, vh, sT)
                return dummy
            return diag_body

        for k in range(N_DEV):
            c = my - k

            @pl.when(c >= 0)
            def _chunk():
                if k == 0:
                    cp = pltpu.make_async_copy(kvp_ref, kvbuf, chunk_sem)
                    cp.start()
                    cp.wait()
                    for qt in range(N_QT):
                        lax.fori_loop(0, H, make_diag_body(qt), 0)
                else:
                    pltpu.make_async_remote_copy(
                        src_ref=kvall_ref.at[c],
                        dst_ref=kvall_ref.at[c],
                        send_sem=send_sems.at[0],
                        recv_sem=recv_sems.at[c],
                        device_id=(my,),
                        device_id_type=pl.DeviceIdType.MESH,
                    ).wait_recv()
                    cp = pltpu.make_async_copy(
                        kvall_ref.at[c], kvbuf, chunk_sem)
                    cp.start()
                    cp.wait()
                    lax.fori_loop(0, N_HQ, make_hq_body(c), 0)

        def ep_body(idx, dummy):
            h = idx // N_QT
            qt = idx % N_QT
            rs = qt * QT
            ctx_ref[pl.ds(h * DH, DH), pl.ds(rs, QT)] = (
                acc_ref[idx] / l_ref[idx]).astype(jnp.bfloat16)
            return dummy

        lax.fori_loop(0, N_HQ, ep_body, 0)
        for rb in range(4):
            rs = rb * (S // 4)
            out_ref[0, rs:rs + S // 4, :] = lax.dot_general(
                ctx_ref[:, rs:rs + S // 4], wo_ref[...],
                (((0,), (0,)), ((), ())),
                preferred_element_type=jnp.float32,
            )

        for off in range(1, N_DEV):
            @pl.when(my + off <= N_DEV - 1)
            def _drain():
                pltpu.make_async_remote_copy(
                    src_ref=kvp_ref,
                    dst_ref=kvall_ref.at[my],
                    send_sem=send_sems.at[off - 1],
                    recv_sem=recv_sems.at[0],
                    device_id=(my,),
                    device_id_type=pl.DeviceIdType.MESH,
                ).wait_send()

    out, _ = pl.pallas_call(
        body,
        out_shape=[
            jax.ShapeDtypeStruct((1, S, D), jnp.float32),
            jax.ShapeDtypeStruct((N_DEV, H, S, 2 * DH), jnp.bfloat16),
        ],
        in_specs=[
            pl.BlockSpec(memory_space=pltpu.MemorySpace.HBM),
            pl.BlockSpec(memory_space=pltpu.MemorySpace.VMEM),
            pl.BlockSpec(memory_space=pltpu.MemorySpace.VMEM),
            pl.BlockSpec(memory_space=pltpu.MemorySpace.VMEM),
        ],
        out_specs=[
            pl.BlockSpec(memory_space=pltpu.MemorySpace.VMEM),
            pl.BlockSpec(memory_space=pltpu.MemorySpace.HBM),
        ],
        scratch_shapes=[
            pltpu.VMEM((H, S, DH), jnp.bfloat16),
            pltpu.VMEM((H, S, 2 * DH), jnp.bfloat16),
            pltpu.VMEM((N_HQ, DH, QT), jnp.float32),
            pltpu.VMEM((N_HQ, 1, QT), jnp.float32),
            pltpu.VMEM((N_HQ, 1, QT), jnp.float32),
            pltpu.VMEM((H * DH, S), jnp.bfloat16),
            pltpu.SemaphoreType.DMA,
            pltpu.SemaphoreType.DMA((N_DEV - 1,)),
            pltpu.SemaphoreType.DMA((N_DEV,)),
        ],
        compiler_params=pltpu.CompilerParams(collective_id=0),
    )(kvp, xb, wqb, wob)
    return out


def kernel(x, Wq, K_ext, V_ext, Wo):
    xb = x[0].astype(jnp.bfloat16)
    wqb = Wq.astype(jnp.bfloat16)
    wob = Wo.astype(jnp.bfloat16)
    kvp = jnp.concatenate(
        [K_ext[0].transpose(1, 0, 2).astype(jnp.bfloat16),
         V_ext[0].transpose(1, 0, 2).astype(jnp.bfloat16)],
        axis=-1,
    )
    return _fused(kvp, xb, wqb, wob)
